# baseline (speedup 1.0000x reference)
import functools

import jax
import jax.numpy as jnp
from jax import lax
from jax.experimental import pallas as pl
from jax.experimental.pallas import tpu as pltpu
from jax.experimental.pallas import tpu_sc as plsc

N = 10000
D = 128
E = 320000

NUM_CORES = 2
NUM_SUBCORES = 16
NUM_WORKERS = NUM_CORES * NUM_SUBCORES
EDGES_PER_WORKER = E // NUM_WORKERS
CHUNK = 128
EDGES_PAD = 10240
CHUNKS_PER_WORKER = EDGES_PAD // CHUNK
N_PAD = 10240
ROWS_PER_TILE = N_PAD // NUM_SUBCORES
WB_ITERS = ROWS_PER_TILE // CHUNK


def _sc_agg_kernel(x_hbm, src_hbm, dst_hbm, out_hbm,
                   src_idx, dst_idx, rows0, agg_sh):
    c = lax.axis_index("c")
    s = lax.axis_index("s")
    wid = c * NUM_SUBCORES + s

    def zrow(i, _):
        for j in range(D // 16):
            rows0[i, pl.ds(j * 16, 16)] = jnp.zeros((16,), jnp.float32)
        return 0
    lax.fori_loop(0, CHUNK, zrow, 0)

    def zcopy(i, _):
        pltpu.sync_copy(rows0,
                        agg_sh.at[pl.ds(s * ROWS_PER_TILE + i * CHUNK, CHUNK)])
        return 0
    lax.fori_loop(0, WB_ITERS, zcopy, 0)

    pltpu.sync_copy(src_hbm.at[wid], src_idx)
    pltpu.sync_copy(dst_hbm.at[wid], dst_idx)

    plsc.subcore_barrier()

    def ebody(t, _):
        pltpu.sync_copy(x_hbm.at[src_idx.at[t]], rows0)
        pltpu.sync_copy(rows0, agg_sh.at[dst_idx.at[t]], add=True)
        return 0
    lax.fori_loop(0, CHUNKS_PER_WORKER, ebody, 0)

    plsc.subcore_barrier()

    def wb(i, _):
        sl = pl.ds(s * ROWS_PER_TILE + i * CHUNK, CHUNK)
        pltpu.sync_copy(agg_sh.at[sl], out_hbm.at[c].at[sl])
        return 0
    lax.fori_loop(0, WB_ITERS, wb, 0)


def _sc_agg(h, src3, dst3):
    mesh = plsc.VectorSubcoreMesh(core_axis_name="c", subcore_axis_name="s")
    f = functools.partial(
        pl.kernel,
        mesh=mesh,
        out_type=jax.ShapeDtypeStruct((NUM_CORES, N_PAD, D), jnp.float32),
        scratch_types=[
            pltpu.VMEM((CHUNKS_PER_WORKER, CHUNK), jnp.int32),
            pltpu.VMEM((CHUNKS_PER_WORKER, CHUNK), jnp.int32),
            pltpu.VMEM((CHUNK, D), jnp.float32),
            pltpu.VMEM_SHARED((N_PAD, D), jnp.float32),
        ],
    )(_sc_agg_kernel)
    p = f(h, src3, dst3)
    return p[0, :N], p[1, :N]


def _tc_layer_kernel(h_ref, p0_ref, p1_ref, w_ref, b_ref, g_ref, bt_ref, o_ref):
    hs = h_ref[...] + p0_ref[...] + p1_ref[...]
    z = lax.dot_general(hs, w_ref[...], (((1,), (1,)), ((), ())),
                        preferred_element_type=jnp.float32) + b_ref[...]
    r = jnp.maximum(z, 0.0)
    mu = jnp.mean(r, axis=0, keepdims=True)
    var = jnp.mean((r - mu) ** 2, axis=0, keepdims=True)
    o_ref[...] = (r - mu) * lax.rsqrt(var + 1e-5) * g_ref[...] + bt_ref[...]


def _tc_layer(h, p0, p1, w, b, g, bt):
    return pl.pallas_call(
        _tc_layer_kernel,
        out_shape=jax.ShapeDtypeStruct((N, D), jnp.float32),
    )(h, p0, p1, w, b.reshape(1, D), g.reshape(1, D), bt.reshape(1, D))


def _tc_final_kernel(h_ref, p0_ref, p1_ref, w_ref, b_ref, o_ref):
    hs = h_ref[...] + p0_ref[...] + p1_ref[...]
    z = lax.dot_general(hs, w_ref[...], (((1,), (1,)), ((), ())),
                        preferred_element_type=jnp.float32) + b_ref[...]
    m = jnp.max(z, axis=1, keepdims=True)
    e = jnp.exp(z - m)
    lse = jnp.log(jnp.sum(e, axis=1, keepdims=True)) + m
    o_ref[...] = z - lse


def _tc_final(h, p0, p1, w, b):
    c = w.shape[0]
    return pl.pallas_call(
        _tc_final_kernel,
        out_shape=jax.ShapeDtypeStruct((N, c), jnp.float32),
    )(h, p0, p1, w, b.reshape(1, c))


def _prep_edges(e, fill):
    e2 = e.reshape(NUM_WORKERS, EDGES_PER_WORKER)
    e2 = jnp.pad(e2, ((0, 0), (0, EDGES_PAD - EDGES_PER_WORKER)),
                 constant_values=fill)
    return e2.reshape(NUM_WORKERS, CHUNKS_PER_WORKER, CHUNK)


def kernel(x, edge_index, proj_W, proj_b, W0, b0, W1, b1, W2, b2,
           final_W, final_b, norm_g, norm_b,
           g0, bt0, g1, bt1, g2, bt2):
    src = _prep_edges(edge_index[0], 0)
    dst = _prep_edges(edge_index[1], N)

    h = x
    p0, p1 = _sc_agg(h, src, dst)
    h = _tc_layer(h, p0, p1, proj_W, proj_b, norm_g, norm_b)
    for w, b, g, bt in ((W0, b0, g0, bt0), (W1, b1, g1, bt1),
                        (W2, b2, g2, bt2)):
        p0, p1 = _sc_agg(h, src, dst)
        h = _tc_layer(h, p0, p1, w, b, g, bt)
    p0, p1 = _sc_agg(h, src, dst)
    return _tc_final(h, p0, p1, final_W, final_b)


# speedup vs baseline: 1.0975x; 1.0975x over previous
import functools

import jax
import jax.numpy as jnp
from jax import lax
from jax.experimental import pallas as pl
from jax.experimental.pallas import tpu as pltpu
from jax.experimental.pallas import tpu_sc as plsc

N = 10000
D = 128
E = 320000

NUM_CORES = 2
NUM_SUBCORES = 16
NUM_WORKERS = NUM_CORES * NUM_SUBCORES
EDGES_PER_WORKER = E // NUM_WORKERS
CHUNK = 128
EDGES_PAD = 10240
CHUNKS_PER_WORKER = EDGES_PAD // CHUNK
N_PAD = 10240
ROWS_PER_TILE = N_PAD // NUM_SUBCORES
WB_ITERS = ROWS_PER_TILE // CHUNK
HALF_CHUNKS = CHUNKS_PER_WORKER // 2


def _sc_agg_kernel(x_hbm, src_hbm, dst_hbm, out_hbm,
                   src_idx, dst_idx, buf_a, buf_b, sem_a, sem_b, agg_sh):
    c = lax.axis_index("c")
    s = lax.axis_index("s")
    wid = c * NUM_SUBCORES + s

    def zrow(i, _):
        for j in range(D // 16):
            buf_a[i, pl.ds(j * 16, 16)] = jnp.zeros((16,), jnp.float32)
        return 0
    lax.fori_loop(0, CHUNK, zrow, 0)

    def zcopy(i, _):
        pltpu.sync_copy(buf_a,
                        agg_sh.at[pl.ds(s * ROWS_PER_TILE + i * CHUNK, CHUNK)])
        return 0
    lax.fori_loop(0, WB_ITERS, zcopy, 0)

    plsc.subcore_barrier()

    for h in range(2):
        pltpu.sync_copy(src_hbm.at[wid].at[pl.ds(h * HALF_CHUNKS, HALF_CHUNKS)],
                        src_idx)
        pltpu.sync_copy(dst_hbm.at[wid].at[pl.ds(h * HALF_CHUNKS, HALF_CHUNKS)],
                        dst_idx)
        pltpu.async_copy(x_hbm.at[src_idx.at[0]], buf_a, sem_a)

        def pair(i, _):
            t = 2 * i
            pltpu.make_async_copy(x_hbm.at[src_idx.at[t]], buf_a, sem_a).wait()
            pltpu.async_copy(x_hbm.at[src_idx.at[t + 1]], buf_b, sem_b)
            pltpu.sync_copy(buf_a, agg_sh.at[dst_idx.at[t]], add=True)

            pltpu.make_async_copy(x_hbm.at[src_idx.at[t + 1]], buf_b,
                                  sem_b).wait()

            @pl.when(t + 2 < HALF_CHUNKS)
            def _():
                pltpu.async_copy(x_hbm.at[src_idx.at[t + 2]], buf_a, sem_a)

            pltpu.sync_copy(buf_b, agg_sh.at[dst_idx.at[t + 1]], add=True)
            return 0

        lax.fori_loop(0, HALF_CHUNKS // 2, pair, 0)

    plsc.subcore_barrier()

    def wb(i, _):
        sl = pl.ds(s * ROWS_PER_TILE + i * CHUNK, CHUNK)
        pltpu.sync_copy(agg_sh.at[sl], out_hbm.at[c].at[sl])
        return 0
    lax.fori_loop(0, WB_ITERS, wb, 0)


def _sc_agg(h, src3, dst3):
    mesh = plsc.VectorSubcoreMesh(core_axis_name="c", subcore_axis_name="s")
    f = functools.partial(
        pl.kernel,
        mesh=mesh,
        out_type=jax.ShapeDtypeStruct((NUM_CORES, N_PAD, D), jnp.float32),
        scratch_types=[
            pltpu.VMEM((HALF_CHUNKS, CHUNK), jnp.int32),
            pltpu.VMEM((HALF_CHUNKS, CHUNK), jnp.int32),
            pltpu.VMEM((CHUNK, D), jnp.float32),
            pltpu.VMEM((CHUNK, D), jnp.float32),
            pltpu.SemaphoreType.DMA,
            pltpu.SemaphoreType.DMA,
            pltpu.VMEM_SHARED((N_PAD, D), jnp.float32),
        ],
    )(_sc_agg_kernel)
    p = f(h, src3, dst3)
    return p[0, :N], p[1, :N]


def _tc_layer_kernel(h_ref, p0_ref, p1_ref, w_ref, b_ref, g_ref, bt_ref, o_ref):
    hs = h_ref[...] + p0_ref[...] + p1_ref[...]
    z = lax.dot_general(hs, w_ref[...], (((1,), (1,)), ((), ())),
                        preferred_element_type=jnp.float32) + b_ref[...]
    r = jnp.maximum(z, 0.0)
    mu = jnp.mean(r, axis=0, keepdims=True)
    var = jnp.mean((r - mu) ** 2, axis=0, keepdims=True)
    o_ref[...] = (r - mu) * lax.rsqrt(var + 1e-5) * g_ref[...] + bt_ref[...]


def _tc_layer(h, p0, p1, w, b, g, bt):
    return pl.pallas_call(
        _tc_layer_kernel,
        out_shape=jax.ShapeDtypeStruct((N, D), jnp.float32),
    )(h, p0, p1, w, b.reshape(1, D), g.reshape(1, D), bt.reshape(1, D))


def _tc_final_kernel(h_ref, p0_ref, p1_ref, w_ref, b_ref, o_ref):
    hs = h_ref[...] + p0_ref[...] + p1_ref[...]
    z = lax.dot_general(hs, w_ref[...], (((1,), (1,)), ((), ())),
                        preferred_element_type=jnp.float32) + b_ref[...]
    m = jnp.max(z, axis=1, keepdims=True)
    e = jnp.exp(z - m)
    lse = jnp.log(jnp.sum(e, axis=1, keepdims=True)) + m
    o_ref[...] = z - lse


def _tc_final(h, p0, p1, w, b):
    c = w.shape[0]
    return pl.pallas_call(
        _tc_final_kernel,
        out_shape=jax.ShapeDtypeStruct((N, c), jnp.float32),
    )(h, p0, p1, w, b.reshape(1, c))


def _prep_edges(e, fill):
    e2 = e.reshape(NUM_WORKERS, EDGES_PER_WORKER)
    e2 = jnp.pad(e2, ((0, 0), (0, EDGES_PAD - EDGES_PER_WORKER)),
                 constant_values=fill)
    return e2.reshape(NUM_WORKERS, CHUNKS_PER_WORKER, CHUNK)


def kernel(x, edge_index, proj_W, proj_b, W0, b0, W1, b1, W2, b2,
           final_W, final_b, norm_g, norm_b,
           g0, bt0, g1, bt1, g2, bt2):
    src = _prep_edges(edge_index[0], 0)
    dst = _prep_edges(edge_index[1], N)

    h = x
    p0, p1 = _sc_agg(h, src, dst)
    h = _tc_layer(h, p0, p1, proj_W, proj_b, norm_g, norm_b)
    for w, b, g, bt in ((W0, b0, g0, bt0), (W1, b1, g1, bt1),
                        (W2, b2, g2, bt2)):
        p0, p1 = _sc_agg(h, src, dst)
        h = _tc_layer(h, p0, p1, w, b, g, bt)
    p0, p1 = _sc_agg(h, src, dst)
    return _tc_final(h, p0, p1, final_W, final_b)
